# baseline (speedup 1.0000x reference)
import jax
import jax.numpy as jnp
from jax import lax
from jax.experimental import pallas as pl
from jax.experimental.pallas import tpu as pltpu
from jax.experimental.pallas import tpu_sc as plsc

BATCH = 16384
HIDDEN = 32
NUM_ROWS = 1000000
NC, NS, L = 2, 16, 16
NW = NC * NS
BPW = BATCH // NW
GROUPS = BPW // L


def _pmf_body(user_ids, item_ids, u_t2, v_t2, out,
              uid_v, iid_v, u_cols, v_cols, out_v, sem_u, sem_v):
    wid = lax.axis_index("s") * NC + lax.axis_index("c")
    base = wid * BPW

    pltpu.sync_copy(user_ids.at[pl.ds(base, BPW)], uid_v)
    pltpu.sync_copy(item_ids.at[pl.ds(base, BPW)], iid_v)

    cps = []
    for h in range(HIDDEN):
        cps.append(pltpu.async_copy(u_t2.at[h].at[uid_v],
                                    u_cols.at[h], sem_u))
        cps.append(pltpu.async_copy(v_t2.at[h].at[iid_v],
                                    v_cols.at[h], sem_v))
    for cp in cps:
        cp.wait()

    def g_body(g, carry):
        p = g * L
        acc = jnp.zeros((L,), jnp.float32)
        for h in range(HIDDEN):
            acc = acc + u_cols[h, pl.ds(p, L)] * v_cols[h, pl.ds(p, L)]
        out_v[pl.ds(p, L)] = jnp.maximum(acc, 0.0)
        return carry

    lax.fori_loop(0, GROUPS, g_body, 0)
    pltpu.sync_copy(out_v, out.at[pl.ds(base, BPW)])


@jax.jit
def kernel(user_ids, item_ids, emb_user, emb_item):
    mesh = plsc.VectorSubcoreMesh(core_axis_name="c", subcore_axis_name="s",
                                  num_cores=NC, num_subcores=NS)
    k = pl.kernel(
        _pmf_body,
        out_type=jax.ShapeDtypeStruct((BATCH,), jnp.float32),
        mesh=mesh,
        scratch_types=[
            pltpu.VMEM((BPW,), jnp.int32),
            pltpu.VMEM((BPW,), jnp.int32),
            pltpu.VMEM((HIDDEN, BPW), jnp.float32),
            pltpu.VMEM((HIDDEN, BPW), jnp.float32),
            pltpu.VMEM((BPW,), jnp.float32),
            pltpu.SemaphoreType.DMA,
            pltpu.SemaphoreType.DMA,
        ],
        compiler_params=pltpu.CompilerParams(use_tc_tiling_on_sc=False,
                                             needs_layout_passes=False),
    )
    return k(user_ids.astype(jnp.int32), item_ids.astype(jnp.int32),
             emb_user.T, emb_item.T)


# speedup vs baseline: 19.4414x; 19.4414x over previous
import jax
import jax.numpy as jnp
from jax import lax
from jax.experimental import pallas as pl
from jax.experimental.pallas import tpu as pltpu
from jax.experimental.pallas import tpu_sc as plsc

BATCH = 16384
HIDDEN = 32
NUM_ROWS = 1000000
NC, NS, L = 2, 16, 16
NW = NC * NS
BPW = BATCH // NW
SUBL = 8
HT = HIDDEN // SUBL
CH = 8
NCHUNK = BPW // CH
IDPAD = BPW + L
TW = 128


def _pmf_body(user_ids, item_ids, u_t3, v_t3, out,
              uid_v, iid_v, u_win, v_win, out_v, sem_u, sem_v):
    wid = lax.axis_index("s") * NC + lax.axis_index("c")
    base = wid * BPW

    pltpu.sync_copy(user_ids.at[pl.ds(base, BPW)], uid_v.at[pl.ds(0, BPW)])
    pltpu.sync_copy(item_ids.at[pl.ds(base, BPW)], iid_v.at[pl.ds(0, BPW)])

    lane0 = lax.iota(jnp.int32, L) == 0
    i16 = lax.iota(jnp.int32, L)
    tr_lo = i16 // SUBL
    s_all = i16 % SUBL
    zeros = jnp.zeros((L,), jnp.float32)

    def chunk_body(c, res):
        cps = []
        rs = []
        for j in range(CH):
            i = c * CH + j
            ru = lax.reduce_sum_p.bind(
                jnp.where(lane0, uid_v[pl.ds(i, L)], 0), axes=(0,))
            rv = lax.reduce_sum_p.bind(
                jnp.where(lane0, iid_v[pl.ds(i, L)], 0), axes=(0,))
            tu = pl.multiple_of(
                lax.shift_left(lax.shift_right_logical(ru, 7), 7), TW)
            tv = pl.multiple_of(
                lax.shift_left(lax.shift_right_logical(rv, 7), 7), TW)
            rs.append((ru - tu, rv - tv))
            cps.append(pltpu.async_copy(
                u_t3.at[:, :, pl.ds(tu, TW)],
                u_win.at[:, :, pl.ds(j * TW, TW)], sem_u))
            cps.append(pltpu.async_copy(
                v_t3.at[:, :, pl.ds(tv, TW)],
                v_win.at[:, :, pl.ds(j * TW, TW)], sem_v))
        for cp in cps:
            cp.wait()

        for j in range(CH):
            lu, lv = rs[j]
            dot = zeros
            for half in range(2):
                tr_h = tr_lo + 2 * half
                iu = jnp.full((L,), j * TW, jnp.int32) + lu
                iv = jnp.full((L,), j * TW, jnp.int32) + lv
                gu = plsc.load_gather(u_win, [tr_h, s_all, iu])
                gv = plsc.load_gather(v_win, [tr_h, s_all, iv])
                dot = dot + gu * gv
            d = lax.reduce_sum_p.bind(dot, axes=(0,))
            lane = (c % 2) * CH + j
            res = jnp.where(i16 == lane, d, res)

        @pl.when(c % 2 == 1)
        def _():
            out_v[pl.ds((c // 2) * L, L)] = jnp.maximum(res, 0.0)

        return res

    lax.fori_loop(0, NCHUNK, chunk_body, zeros)
    pltpu.sync_copy(out_v, out.at[pl.ds(base, BPW)])


@jax.jit
def kernel(user_ids, item_ids, emb_user, emb_item):
    mesh = plsc.VectorSubcoreMesh(core_axis_name="c", subcore_axis_name="s",
                                  num_cores=NC, num_subcores=NS)
    k = pl.kernel(
        _pmf_body,
        out_type=jax.ShapeDtypeStruct((BATCH,), jnp.float32),
        mesh=mesh,
        scratch_types=[
            pltpu.VMEM((IDPAD,), jnp.int32),
            pltpu.VMEM((IDPAD,), jnp.int32),
            pltpu.VMEM((HT, SUBL, CH * TW), jnp.float32),
            pltpu.VMEM((HT, SUBL, CH * TW), jnp.float32),
            pltpu.VMEM((BPW,), jnp.float32),
            pltpu.SemaphoreType.DMA,
            pltpu.SemaphoreType.DMA,
        ],
        compiler_params=pltpu.CompilerParams(use_tc_tiling_on_sc=True,
                                             needs_layout_passes=False),
    )
    u_t3 = emb_user.T.reshape(HT, SUBL, NUM_ROWS)
    v_t3 = emb_item.T.reshape(HT, SUBL, NUM_ROWS)
    return k(user_ids.astype(jnp.int32), item_ids.astype(jnp.int32),
             u_t3, v_t3)


# speedup vs baseline: 21.9246x; 1.1277x over previous
import jax
import jax.numpy as jnp
from jax import lax
from jax.experimental import pallas as pl
from jax.experimental.pallas import tpu as pltpu
from jax.experimental.pallas import tpu_sc as plsc

BATCH = 16384
HIDDEN = 32
NUM_ROWS = 1000000
NC, NS, L = 2, 16, 16
NW = NC * NS
BPW = BATCH // NW
SUBL = 8
HT = HIDDEN // SUBL
CH = 4
NCHUNK = BPW // CH
IDPAD = BPW + L
TW = 128


def _pmf_body(user_ids, item_ids, u_t3, v_t3, out,
              uid_v, iid_v, u_win, v_win, out_v,
              sem_u0, sem_v0, sem_u1, sem_v1):
    wid = lax.axis_index("s") * NC + lax.axis_index("c")
    base = wid * BPW
    sem_u = [sem_u0, sem_u1]
    sem_v = [sem_v0, sem_v1]

    pltpu.sync_copy(user_ids.at[pl.ds(base, BPW)], uid_v.at[pl.ds(0, BPW)])
    pltpu.sync_copy(item_ids.at[pl.ds(base, BPW)], iid_v.at[pl.ds(0, BPW)])

    lane0 = lax.iota(jnp.int32, L) == 0
    i16 = lax.iota(jnp.int32, L)
    tr_lo = i16 // SUBL
    s_all = i16 % SUBL
    zeros = jnp.zeros((L,), jnp.float32)

    def issue(c, buf):
        for j in range(CH):
            i = c * CH + j
            ru = lax.reduce_sum_p.bind(
                jnp.where(lane0, uid_v[pl.ds(i, L)], 0), axes=(0,))
            rv = lax.reduce_sum_p.bind(
                jnp.where(lane0, iid_v[pl.ds(i, L)], 0), axes=(0,))
            tu = pl.multiple_of(
                lax.shift_left(lax.shift_right_logical(ru, 7), 7), TW)
            tv = pl.multiple_of(
                lax.shift_left(lax.shift_right_logical(rv, 7), 7), TW)
            s = buf * (CH * TW)
            pltpu.async_copy(u_t3.at[:, :, pl.ds(tu, TW)],
                             u_win.at[:, :, pl.ds(s + j * TW, TW)],
                             sem_u[buf])
            pltpu.async_copy(v_t3.at[:, :, pl.ds(tv, TW)],
                             v_win.at[:, :, pl.ds(s + j * TW, TW)],
                             sem_v[buf])

    def drain(buf):
        s = buf * (CH * TW)
        pltpu.make_async_copy(
            u_t3.at[:, :, pl.ds(0, CH * TW)],
            u_win.at[:, :, pl.ds(s, CH * TW)], sem_u[buf]).wait()
        pltpu.make_async_copy(
            v_t3.at[:, :, pl.ds(0, CH * TW)],
            v_win.at[:, :, pl.ds(s, CH * TW)], sem_v[buf]).wait()

    def compute(c, buf, res):
        for j in range(CH):
            i = c * CH + j
            ru16 = jnp.where(lane0, uid_v[pl.ds(i, L)], 0)
            rv16 = jnp.where(lane0, iid_v[pl.ds(i, L)], 0)
            lu = lax.reduce_sum_p.bind(ru16 & 127, axes=(0,))
            lv = lax.reduce_sum_p.bind(rv16 & 127, axes=(0,))
            off = buf * (CH * TW) + j * TW
            dot = zeros
            for half in range(2):
                tr_h = tr_lo + 2 * half
                iu = jnp.full((L,), off, jnp.int32) + lu
                iv = jnp.full((L,), off, jnp.int32) + lv
                gu = plsc.load_gather(u_win, [tr_h, s_all, iu])
                gv = plsc.load_gather(v_win, [tr_h, s_all, iv])
                dot = dot + gu * gv
            d = lax.reduce_sum_p.bind(dot, axes=(0,))
            res = jnp.where(i16 == (c % 4) * CH + j, d, res)
        return res

    issue(0, 0)

    def pipe(k, res):
        c0 = 2 * k
        issue(c0 + 1, 1)
        drain(0)
        res = compute(c0, 0, res)

        @pl.when(c0 + 2 < NCHUNK)
        def _():
            issue(c0 + 2, 0)

        drain(1)
        res = compute(c0 + 1, 1, res)

        @pl.when(k % 2 == 1)
        def _():
            out_v[pl.ds(((c0 + 1) // 4) * L, L)] = jnp.maximum(res, 0.0)

        return res

    lax.fori_loop(0, NCHUNK // 2, pipe, zeros)
    pltpu.sync_copy(out_v, out.at[pl.ds(base, BPW)])


@jax.jit
def kernel(user_ids, item_ids, emb_user, emb_item):
    mesh = plsc.VectorSubcoreMesh(core_axis_name="c", subcore_axis_name="s",
                                  num_cores=NC, num_subcores=NS)
    k = pl.kernel(
        _pmf_body,
        out_type=jax.ShapeDtypeStruct((BATCH,), jnp.float32),
        mesh=mesh,
        scratch_types=[
            pltpu.VMEM((IDPAD,), jnp.int32),
            pltpu.VMEM((IDPAD,), jnp.int32),
            pltpu.VMEM((HT, SUBL, 2 * CH * TW), jnp.float32),
            pltpu.VMEM((HT, SUBL, 2 * CH * TW), jnp.float32),
            pltpu.VMEM((BPW,), jnp.float32),
            pltpu.SemaphoreType.DMA,
            pltpu.SemaphoreType.DMA,
            pltpu.SemaphoreType.DMA,
            pltpu.SemaphoreType.DMA,
        ],
        compiler_params=pltpu.CompilerParams(use_tc_tiling_on_sc=True,
                                             needs_layout_passes=False),
    )
    u_t3 = emb_user.T.reshape(HT, SUBL, NUM_ROWS)
    v_t3 = emb_item.T.reshape(HT, SUBL, NUM_ROWS)
    return k(user_ids.astype(jnp.int32), item_ids.astype(jnp.int32),
             u_t3, v_t3)
